# baseline (speedup 1.0000x reference)
import functools

import jax
import jax.numpy as jnp
from jax import lax
from jax.experimental import pallas as pl
from jax.experimental.pallas import tpu as pltpu
from jax.experimental.pallas import tpu_sc as plsc

_B = 16384
_D = 128
_CHUNK = 128

_NBUF = 6
_LAG = 2


def _build():
    info = plsc.get_sparse_core_info()
    nc, ns = info.num_cores, info.num_subcores
    nw = nc * ns
    b_per_w = _B // nw
    n_chunks = b_per_w // _CHUNK
    n_t = 3 * n_chunks
    mesh = plsc.VectorSubcoreMesh(core_axis_name="c", subcore_axis_name="s")
    out_t = jax.ShapeDtypeStruct((_B, _D), jnp.float32)

    @functools.partial(
        pl.kernel,
        out_type=(out_t, out_t, out_t),
        mesh=mesh,
        scratch_types=[
            pltpu.VMEM((n_t, _CHUNK), jnp.int32),
        ] + [pltpu.VMEM((_CHUNK, _D), jnp.float32)] * _NBUF
          + [pltpu.SemaphoreType.DMA] * (2 * _NBUF),
    )
    def k(idx_hbm, ent_hbm, rel_hbm, out_qe, out_qr, out_oe, idx_v, *rest):
        bufs = rest[:_NBUF]
        gsems = rest[_NBUF:2 * _NBUF]
        wsems = rest[2 * _NBUF:]
        wid = lax.axis_index("s") * nc + lax.axis_index("c")
        pltpu.sync_copy(idx_hbm.at[wid], idx_v)

        tables = (ent_hbm, rel_hbm, ent_hbm)
        outs = (out_qe, out_qr, out_oe)
        base = wid * b_per_w

        def gather(t):
            return pltpu.make_async_copy(
                tables[divmod(t, n_chunks)[0]].at[idx_v.at[t]],
                bufs[t % _NBUF], gsems[t % _NBUF])

        def writeout(t):
            g, j = divmod(t, n_chunks)
            return pltpu.make_async_copy(
                bufs[t % _NBUF],
                outs[g].at[pl.ds(base + j * _CHUNK, _CHUNK)],
                wsems[t % _NBUF])

        for t in range(_NBUF):
            gather(t).start()
        for t in range(n_t):
            gather(t).wait()
            writeout(t).start()
            s = t - _LAG
            if s >= 0 and s + _NBUF < n_t:
                writeout(s).wait()
                gather(s + _NBUF).start()
        for t in range(n_t - _NBUF, n_t):
            writeout(t).wait()

    return k


_kernel_fn = _build()


def kernel(query_entities, query_relations, obj_entities, ent_table, rel_table):
    nw = 32
    per_w = _B // nw // _CHUNK
    idx = jnp.stack([
        query_entities.reshape(nw, per_w, _CHUNK),
        query_relations.reshape(nw, per_w, _CHUNK),
        obj_entities.reshape(nw, per_w, _CHUNK),
    ], axis=1).reshape(nw, 3 * per_w, _CHUNK)
    return _kernel_fn(idx, ent_table, rel_table)


# speedup vs baseline: 1.0030x; 1.0030x over previous
import functools

import jax
import jax.numpy as jnp
from jax import lax
from jax.experimental import pallas as pl
from jax.experimental.pallas import tpu as pltpu
from jax.experimental.pallas import tpu_sc as plsc

_B = 16384
_D = 128
_CHUNK = 128

_NBUF = 6
_LAG = 2


def _build():
    info = plsc.get_sparse_core_info()
    nc, ns = info.num_cores, info.num_subcores
    nw = nc * ns
    b_per_w = _B // nw
    n_chunks = b_per_w // _CHUNK
    n_t = 3 * n_chunks
    mesh = plsc.VectorSubcoreMesh(core_axis_name="c", subcore_axis_name="s")
    out_t = jax.ShapeDtypeStruct((_B, _D), jnp.float32)

    @functools.partial(
        pl.kernel,
        out_type=(out_t, out_t, out_t),
        mesh=mesh,
        scratch_types=[
            pltpu.VMEM((n_chunks, _CHUNK), jnp.int32),
            pltpu.VMEM((n_chunks, _CHUNK), jnp.int32),
            pltpu.VMEM((n_chunks, _CHUNK), jnp.int32),
            pltpu.SemaphoreType.DMA,
            pltpu.SemaphoreType.DMA,
            pltpu.SemaphoreType.DMA,
        ] + [pltpu.VMEM((_CHUNK, _D), jnp.float32)] * _NBUF
          + [pltpu.SemaphoreType.DMA] * (2 * _NBUF),
    )
    def k(iqe_hbm, iqr_hbm, ioe_hbm, ent_hbm, rel_hbm,
          out_qe, out_qr, out_oe, iv0, iv1, iv2, is0, is1, is2, *rest):
        bufs = rest[:_NBUF]
        gsems = rest[_NBUF:2 * _NBUF]
        wsems = rest[2 * _NBUF:]
        wid = lax.axis_index("s") * nc + lax.axis_index("c")
        idx_v = (iv0, iv1, iv2)
        isems = (is0, is1, is2)
        icpy = [pltpu.make_async_copy(src.at[wid], dst, sem)
                for src, dst, sem in zip((iqe_hbm, iqr_hbm, ioe_hbm),
                                         idx_v, isems)]
        for c in icpy:
            c.start()

        tables = (ent_hbm, rel_hbm, ent_hbm)
        outs = (out_qe, out_qr, out_oe)
        base = wid * b_per_w
        idx_ready = [False, False, False]

        def gather(t):
            g, j = divmod(t, n_chunks)
            if not idx_ready[g]:
                icpy[g].wait()
                idx_ready[g] = True
            return pltpu.make_async_copy(
                tables[g].at[idx_v[g].at[j]],
                bufs[t % _NBUF], gsems[t % _NBUF])

        def writeout(t):
            g, j = divmod(t, n_chunks)
            return pltpu.make_async_copy(
                bufs[t % _NBUF],
                outs[g].at[pl.ds(base + j * _CHUNK, _CHUNK)],
                wsems[t % _NBUF])

        for t in range(_NBUF):
            gather(t).start()
        for t in range(n_t):
            gather(t).wait()
            writeout(t).start()
            s = t - _LAG
            if s >= 0 and s + _NBUF < n_t:
                writeout(s).wait()
                gather(s + _NBUF).start()
        for t in range(n_t - _NBUF, n_t):
            writeout(t).wait()

    return k


_kernel_fn = _build()


def kernel(query_entities, query_relations, obj_entities, ent_table, rel_table):
    nw = 32
    per_w = _B // nw // _CHUNK
    shp = (nw, per_w, _CHUNK)
    return _kernel_fn(query_entities.reshape(shp),
                      query_relations.reshape(shp),
                      obj_entities.reshape(shp),
                      ent_table, rel_table)
